# baseline (speedup 1.0000x reference)
import functools

import jax
import jax.numpy as jnp
from jax import lax
from jax.experimental import pallas as pl
from jax.experimental.pallas import tpu as pltpu
from jax.experimental.pallas import tpu_sc as plsc

N = 100000
E = 3200000

NC = 2
NS = 16
NW = NC * NS
CHUNK = 128
KU = 4
KU_D = 8
RPT = 784
R = NW * RPT
E_PAD = R * CHUNK
NP_ = 100096
DSL = NP_ // NS
NACC = NP_
NP_D = 102400
DSL_D = NP_D // NS

_MESH = plsc.VectorSubcoreMesh(core_axis_name="c", subcore_axis_name="s")
_SC_PARAMS = pltpu.CompilerParams(use_tc_tiling_on_sc=False)


def _make_deg():
  @functools.partial(
      pl.kernel,
      out_type=jax.ShapeDtypeStruct((NC, 1, NP_D), jnp.float32),
      mesh=_MESH,
      scratch_types=[
          pltpu.VMEM((2, KU_D, 1, CHUNK), jnp.int32),
          pltpu.VMEM((CHUNK,), jnp.float32),
          pltpu.VMEM_SHARED((NP_D,), jnp.float32),
          pltpu.SemaphoreType.DMA,
      ],
      compiler_params=_SC_PARAMS,
  )
  def deg_kernel(edges, ones, zeros, out, dstv, onesv, acc, ssem):
    c = lax.axis_index("c")
    s = lax.axis_index("s")
    wid = c * NS + s
    pltpu.sync_copy(zeros.at[pl.ds(s * DSL_D, DSL_D)],
                    acc.at[pl.ds(s * DSL_D, DSL_D)])
    pltpu.sync_copy(ones, onesv)
    plsc.subcore_barrier()
    base = wid * RPT

    @pl.loop(0, RPT // KU_D, step=2)
    def _(i):
      r0 = base + i * KU_D
      sds = []
      for b in range(2):
        pltpu.sync_copy(edges.at[pl.ds(r0 + b * KU_D, KU_D), pl.ds(1, 1)],
                        dstv.at[b])
        for k in range(KU_D):
          sds.append(pltpu.async_copy(onesv, acc.at[dstv.at[b, k, 0]], ssem,
                                      add=True))
      for d in sds:
        d.wait()

    plsc.subcore_barrier()
    pltpu.sync_copy(acc.at[pl.ds(s * DSL_D, DSL_D)],
                    out.at[c, 0, pl.ds(s * DSL_D, DSL_D)])

  return deg_kernel


def _make_agg(fh):
  @functools.partial(
      pl.kernel,
      out_type=jax.ShapeDtypeStruct((NC, NP_, fh), jnp.float32),
      mesh=_MESH,
      scratch_types=[
          pltpu.VMEM((2, KU, 2, CHUNK), jnp.int32),
          pltpu.VMEM((2, KU, CHUNK, fh), jnp.float32),
          pltpu.VMEM_SHARED((NACC, fh), jnp.float32),
          pltpu.SemaphoreType.DMA,
          pltpu.SemaphoreType.DMA,
          pltpu.SemaphoreType.DMA,
      ],
      compiler_params=_SC_PARAMS,
  )
  def agg_kernel(edges, table, zeros, out, eidx, gbuf, acc, gsem0, gsem1,
                 ssem):
    gsems = (gsem0, gsem1)
    c = lax.axis_index("c")
    s = lax.axis_index("s")
    wid = c * NS + s
    pltpu.sync_copy(zeros.at[pl.ds(s * DSL, DSL)], acc.at[pl.ds(s * DSL, DSL)])
    plsc.subcore_barrier()
    base = wid * RPT

    @pl.loop(0, RPT // KU, step=2)
    def _(i):
      r0 = base + i * KU
      gds = [None, None]
      for b in range(2):
        pltpu.sync_copy(edges.at[pl.ds(r0 + b * KU, KU)], eidx.at[b])
        gds[b] = [
            pltpu.async_copy(table.at[eidx.at[b, k, 0]], gbuf.at[b, k],
                             gsems[b])
            for k in range(KU)
        ]
      sds = []
      for b in range(2):
        for k in range(KU):
          gds[b][k].wait()
        for k in range(KU):
          sds.append(pltpu.async_copy(
              gbuf.at[b, k], acc.at[eidx.at[b, k, 1]], ssem, add=True))
      for d in sds:
        d.wait()

    plsc.subcore_barrier()
    pltpu.sync_copy(acc.at[pl.ds(s * DSL, DSL)], out.at[c, pl.ds(s * DSL, DSL)])

  return agg_kernel


_deg = _make_deg()
_agg16 = _make_agg(16)
_agg8 = _make_agg(8)


_B = 5000
_G = N // _B


def _row_spec(f):
  return pl.BlockSpec((_B, f), lambda i: (i, 0))


def _full_spec(shape):
  nd = len(shape)
  return pl.BlockSpec(shape, lambda i: (0,) * nd)


def _agg_spec(f):
  return pl.BlockSpec((NC, _B, f), lambda i: (0, i, 0))


def _pre_body(x, w, dis, oa, ob):
  h = jnp.dot(x[...], w[...], preferred_element_type=jnp.float32)
  h = h * dis[...]
  oa[...] = h[:, :16]
  ob[...] = h[:, 16:]


def _k_pre(x, w1, dis):
  return pl.pallas_call(
      _pre_body,
      grid=(_G,),
      in_specs=[_row_spec(7), _full_spec((7, 32)), _row_spec(1)],
      out_specs=[_row_spec(16), _row_spec(16)],
      out_shape=[
          jax.ShapeDtypeStruct((N, 16), jnp.float32),
          jax.ShapeDtypeStruct((N, 16), jnp.float32),
      ],
  )(x, w1, dis)


def _mid_body(p, hh, dis, b, o, st):
  i = pl.program_id(0)
  y = dis[...] * (p[0] + p[1] + hh[...]) + b[...]
  o[...] = y

  @pl.when(i == 0)
  def _():
    st[...] = jnp.zeros_like(st)

  st[...] += jnp.stack([jnp.sum(y, axis=0), jnp.sum(y * y, axis=0)])


def _k_mid(p, hh, dis, b, f):
  return pl.pallas_call(
      _mid_body,
      grid=(_G,),
      in_specs=[_agg_spec(f), _row_spec(f),
                _row_spec(1), _full_spec((1, f))],
      out_specs=[_row_spec(f), _full_spec((2, f))],
      out_shape=[
          jax.ShapeDtypeStruct((N, f), jnp.float32),
          jax.ShapeDtypeStruct((2, f), jnp.float32),
      ],
  )(p, hh, dis, b)


def _post2_body(oa, ob, sta, stb, g, be, dis, w, hn):
  o = jnp.concatenate([oa[...], ob[...]], axis=1)
  st = jnp.concatenate([sta[...], stb[...]], axis=1)
  m = st[0:1, :] * (1.0 / N)
  v = st[1:2, :] * (1.0 / N) - m * m
  h = jnp.maximum((o - m) * lax.rsqrt(v + 1e-5) * g[...] + be[...], 0.0)
  hn[...] = jnp.dot(h, w[...], preferred_element_type=jnp.float32) * dis[...]


def _k_post2(oa, ob, sta, stb, g, be, dis, w):
  return pl.pallas_call(
      _post2_body,
      grid=(_G,),
      in_specs=[
          _row_spec(16),
          _row_spec(16),
          _full_spec((2, 16)),
          _full_spec((2, 16)),
          _full_spec((1, 32)),
          _full_spec((1, 32)),
          _row_spec(1),
          _full_spec((32, 16)),
      ],
      out_specs=_row_spec(16),
      out_shape=jax.ShapeDtypeStruct((N, 16), jnp.float32),
  )(oa, ob, sta, stb, g, be, dis, w)


def _post_body(o, st, g, be, dis, w, hn):
  m = st[0:1, :] * (1.0 / N)
  v = st[1:2, :] * (1.0 / N) - m * m
  h = jnp.maximum((o[...] - m) * lax.rsqrt(v + 1e-5) * g[...] + be[...], 0.0)
  hn[...] = jnp.dot(h, w[...], preferred_element_type=jnp.float32) * dis[...]


def _k_post(o, st, g, be, dis, w, f_in, f_out):
  return pl.pallas_call(
      _post_body,
      grid=(_G,),
      in_specs=[
          _row_spec(f_in),
          _full_spec((2, f_in)),
          _full_spec((1, f_in)),
          _full_spec((1, f_in)),
          _row_spec(1),
          _full_spec((f_in, f_out)),
      ],
      out_specs=_row_spec(f_out),
      out_shape=jax.ShapeDtypeStruct((N, f_out), jnp.float32),
  )(o, st, g, be, dis, w)


def _fin_body(p, hh, dis, b, wc, bc, wr, br, op, orr):
  h3 = jnp.maximum(dis[...] * (p[0] + p[1] + hh[...]) + b[...], 0.0)
  zc = jnp.dot(h3, wc[...], preferred_element_type=jnp.float32) + bc[...]
  op[...] = jax.nn.sigmoid(zc)
  zr = jnp.dot(h3, wr[...], preferred_element_type=jnp.float32) + br[...]
  orr[...] = jnp.maximum(zr, 0.0)


def _k_fin(p, hh, dis, b, wc, bc, wr, br):
  return pl.pallas_call(
      _fin_body,
      grid=(_G,),
      in_specs=[_agg_spec(8), _row_spec(8),
                _row_spec(1), _full_spec((1, 8)),
         _full_spec((8, 1)), _full_spec((1, 1)),
         _full_spec((8, 1)), _full_spec((1, 1))],
      out_specs=[_row_spec(1), _row_spec(1)],
      out_shape=[
          jax.ShapeDtypeStruct((N, 1), jnp.float32),
          jax.ShapeDtypeStruct((N, 1), jnp.float32),
      ],
  )(p, hh, dis, b, wc, bc, wr, br)


def kernel(x, edge_index, W1, b1, g1, be1, W2, b2, g2, be2, W3, b3,
           Wc, bc, Wr, br):
  src = edge_index[0].astype(jnp.int32)
  dst = edge_index[1].astype(jnp.int32)
  pad = E_PAD - E
  srcp = jnp.concatenate([src, jnp.zeros((pad,), jnp.int32)])
  dstp = jnp.concatenate([dst, jnp.full((pad,), N, jnp.int32)])
  edges = jnp.concatenate(
      [srcp.reshape(R, 1, CHUNK), dstp.reshape(R, 1, CHUNK)], axis=1)

  ones = jnp.ones((CHUNK,), jnp.float32)
  z1 = jnp.zeros((NP_D,), jnp.float32)
  z16 = jnp.zeros((NP_, 16), jnp.float32)
  z8 = jnp.zeros((NP_, 8), jnp.float32)

  deg = _deg(edges, ones, z1)
  dtot = deg[0, 0, :N] + deg[1, 0, :N]
  dis = lax.rsqrt(dtot + 1.0).reshape(N, 1)

  b1r = b1.reshape(1, 32)
  g1r = g1.reshape(1, 32)
  be1r = be1.reshape(1, 32)
  b2r = b2.reshape(1, 16)
  g2r = g2.reshape(1, 16)
  be2r = be2.reshape(1, 16)
  b3r = b3.reshape(1, 8)
  bcr = bc.reshape(1, 1)
  brr = br.reshape(1, 1)

  ha, hb = _k_pre(x, W1, dis)
  aa = _agg16(edges, ha, z16)
  ab = _agg16(edges, hb, z16)
  o1a, st1a = _k_mid(aa, ha, dis, b1r[:, :16], 16)
  o1b, st1b = _k_mid(ab, hb, dis, b1r[:, 16:], 16)
  h2 = _k_post2(o1a, o1b, st1a, st1b, g1r, be1r, dis, W2)
  a2 = _agg16(edges, h2, z16)
  o2, st2 = _k_mid(a2, h2, dis, b2r, 16)
  h3 = _k_post(o2, st2, g2r, be2r, dis, W3, 16, 8)
  a3 = _agg8(edges, h3, z8)
  p, r = _k_fin(a3, h3, dis, b3r, Wc, bcr, Wr, brr)
  return (p, r)


# speedup vs baseline: 1.0291x; 1.0291x over previous
import functools

import jax
import jax.numpy as jnp
from jax import lax
from jax.experimental import pallas as pl
from jax.experimental.pallas import tpu as pltpu
from jax.experimental.pallas import tpu_sc as plsc

N = 100000
E = 3200000

NC = 2
NS = 16
NW = NC * NS
CHUNK = 128
KU = 4
KU_D = 8
RPT = 784
R = NW * RPT
E_PAD = R * CHUNK
NP_ = 100096
DSL = NP_ // NS
NACC = NP_
NP_D = 102400
DSL_D = NP_D // NS

_MESH = plsc.VectorSubcoreMesh(core_axis_name="c", subcore_axis_name="s")
_SC_PARAMS = pltpu.CompilerParams(use_tc_tiling_on_sc=False)


def _make_deg():
  @functools.partial(
      pl.kernel,
      out_type=jax.ShapeDtypeStruct((NC, 1, NP_D), jnp.float32),
      mesh=_MESH,
      scratch_types=[
          pltpu.VMEM((2, KU_D, 1, CHUNK), jnp.int32),
          pltpu.VMEM((CHUNK,), jnp.float32),
          pltpu.VMEM_SHARED((NP_D,), jnp.float32),
          pltpu.SemaphoreType.DMA,
      ],
      compiler_params=_SC_PARAMS,
  )
  def deg_kernel(edges, ones, zeros, out, dstv, onesv, acc, ssem):
    c = lax.axis_index("c")
    s = lax.axis_index("s")
    wid = c * NS + s
    pltpu.sync_copy(zeros.at[pl.ds(s * DSL_D, DSL_D)],
                    acc.at[pl.ds(s * DSL_D, DSL_D)])
    pltpu.sync_copy(ones, onesv)
    plsc.subcore_barrier()
    base = wid * RPT

    @pl.loop(0, RPT // KU_D, step=2)
    def _(i):
      r0 = base + i * KU_D
      sds = []
      for b in range(2):
        pltpu.sync_copy(edges.at[pl.ds(r0 + b * KU_D, KU_D), pl.ds(1, 1)],
                        dstv.at[b])
        for k in range(KU_D):
          sds.append(pltpu.async_copy(onesv, acc.at[dstv.at[b, k, 0]], ssem,
                                      add=True))
      for d in sds:
        d.wait()

    plsc.subcore_barrier()
    pltpu.sync_copy(acc.at[pl.ds(s * DSL_D, DSL_D)],
                    out.at[c, 0, pl.ds(s * DSL_D, DSL_D)])

  return deg_kernel


def _make_agg(fh):
  @functools.partial(
      pl.kernel,
      out_type=jax.ShapeDtypeStruct((NC, NP_, fh), jnp.float32),
      mesh=_MESH,
      scratch_types=[
          pltpu.VMEM((2, KU, 2, CHUNK), jnp.int32),
          pltpu.VMEM((2, KU, CHUNK, fh), jnp.float32),
          pltpu.VMEM_SHARED((NACC, fh), jnp.float32),
          pltpu.SemaphoreType.DMA,
          pltpu.SemaphoreType.DMA,
          pltpu.SemaphoreType.DMA,
      ],
      compiler_params=_SC_PARAMS,
  )
  def agg_kernel(edges, table, zeros, out, eidx, gbuf, acc, gsem0, gsem1,
                 ssem):
    gsems = (gsem0, gsem1)
    c = lax.axis_index("c")
    s = lax.axis_index("s")
    wid = c * NS + s
    pltpu.sync_copy(zeros.at[pl.ds(s * DSL, DSL)], acc.at[pl.ds(s * DSL, DSL)])
    plsc.subcore_barrier()
    base = wid * RPT

    @pl.loop(0, RPT // KU, step=2)
    def _(i):
      r0 = base + i * KU
      gds = [None, None]
      for b in range(2):
        pltpu.sync_copy(edges.at[pl.ds(r0 + b * KU, KU)], eidx.at[b])
        gds[b] = [
            pltpu.async_copy(table.at[eidx.at[b, k, 0]], gbuf.at[b, k],
                             gsems[b])
            for k in range(KU)
        ]
      sds = []
      for b in range(2):
        for k in range(KU):
          gds[b][k].wait()
        for k in range(KU):
          sds.append(pltpu.async_copy(
              gbuf.at[b, k], acc.at[eidx.at[b, k, 1]], ssem, add=True))
      for d in sds:
        d.wait()

    plsc.subcore_barrier()
    pltpu.sync_copy(acc.at[pl.ds(s * DSL, DSL)], out.at[c, pl.ds(s * DSL, DSL)])

  return agg_kernel


_deg = _make_deg()
_agg16 = _make_agg(16)
_agg8 = _make_agg(8)


_B = 5000
_G = N // _B


def _row_spec(f):
  return pl.BlockSpec((_B, f), lambda i: (i, 0))


def _full_spec(shape):
  nd = len(shape)
  return pl.BlockSpec(shape, lambda i: (0,) * nd)


def _agg_spec(f):
  return pl.BlockSpec((NC, _B, f), lambda i: (0, i, 0))


def _pre_body(x, w, dis, oa, ob):
  h = jnp.dot(x[...], w[...], preferred_element_type=jnp.float32)
  h = h * dis[...]
  oa[...] = h[:, :16]
  ob[...] = h[:, 16:]


def _k_pre(x, w1, dis):
  return pl.pallas_call(
      _pre_body,
      grid=(_G,),
      in_specs=[_row_spec(7), _full_spec((7, 32)), _row_spec(1)],
      out_specs=[_row_spec(16), _row_spec(16)],
      out_shape=[
          jax.ShapeDtypeStruct((N, 16), jnp.float32),
          jax.ShapeDtypeStruct((N, 16), jnp.float32),
      ],
  )(x, w1, dis)


def _mid_body(p, hh, dis, b, o, st):
  i = pl.program_id(0)
  y = dis[...] * (p[0] + p[1] + hh[...]) + b[...]
  o[...] = y

  @pl.when(i == 0)
  def _():
    st[...] = jnp.zeros_like(st)

  st[...] += jnp.stack([jnp.sum(y, axis=0), jnp.sum(y * y, axis=0)])


def _k_mid(p, hh, dis, b, f):
  return pl.pallas_call(
      _mid_body,
      grid=(_G,),
      in_specs=[_agg_spec(f), _row_spec(f),
                _row_spec(1), _full_spec((1, f))],
      out_specs=[_row_spec(f), _full_spec((2, f))],
      out_shape=[
          jax.ShapeDtypeStruct((N, f), jnp.float32),
          jax.ShapeDtypeStruct((2, f), jnp.float32),
      ],
  )(p, hh, dis, b)


def _mid1_body(aa, ha, ab, hb, dis, b, o, st):
  i = pl.program_id(0)
  d = dis[...]
  ya = d * (aa[0] + aa[1] + ha[...])
  yb = d * (ab[0] + ab[1] + hb[...])
  y = jnp.concatenate([ya, yb], axis=1) + b[...]
  o[...] = y

  @pl.when(i == 0)
  def _():
    st[...] = jnp.zeros_like(st)

  st[...] += jnp.stack([jnp.sum(y, axis=0), jnp.sum(y * y, axis=0)])


def _k_mid1(aa, ha, ab, hb, dis, b):
  return pl.pallas_call(
      _mid1_body,
      grid=(_G,),
      in_specs=[_agg_spec(16), _row_spec(16), _agg_spec(16), _row_spec(16),
                _row_spec(1), _full_spec((1, 32))],
      out_specs=[_row_spec(32), _full_spec((2, 32))],
      out_shape=[
          jax.ShapeDtypeStruct((N, 32), jnp.float32),
          jax.ShapeDtypeStruct((2, 32), jnp.float32),
      ],
  )(aa, ha, ab, hb, dis, b)


def _post_body(o, st, g, be, dis, w, hn):
  m = st[0:1, :] * (1.0 / N)
  v = st[1:2, :] * (1.0 / N) - m * m
  h = jnp.maximum((o[...] - m) * lax.rsqrt(v + 1e-5) * g[...] + be[...], 0.0)
  hn[...] = jnp.dot(h, w[...], preferred_element_type=jnp.float32) * dis[...]


def _k_post(o, st, g, be, dis, w, f_in, f_out):
  return pl.pallas_call(
      _post_body,
      grid=(_G,),
      in_specs=[
          _row_spec(f_in),
          _full_spec((2, f_in)),
          _full_spec((1, f_in)),
          _full_spec((1, f_in)),
          _row_spec(1),
          _full_spec((f_in, f_out)),
      ],
      out_specs=_row_spec(f_out),
      out_shape=jax.ShapeDtypeStruct((N, f_out), jnp.float32),
  )(o, st, g, be, dis, w)


def _fin_body(p, hh, dis, b, wc, bc, wr, br, op, orr):
  h3 = jnp.maximum(dis[...] * (p[0] + p[1] + hh[...]) + b[...], 0.0)
  zc = jnp.dot(h3, wc[...], preferred_element_type=jnp.float32) + bc[...]
  op[...] = jax.nn.sigmoid(zc)
  zr = jnp.dot(h3, wr[...], preferred_element_type=jnp.float32) + br[...]
  orr[...] = jnp.maximum(zr, 0.0)


def _k_fin(p, hh, dis, b, wc, bc, wr, br):
  return pl.pallas_call(
      _fin_body,
      grid=(_G,),
      in_specs=[_agg_spec(8), _row_spec(8),
                _row_spec(1), _full_spec((1, 8)),
         _full_spec((8, 1)), _full_spec((1, 1)),
         _full_spec((8, 1)), _full_spec((1, 1))],
      out_specs=[_row_spec(1), _row_spec(1)],
      out_shape=[
          jax.ShapeDtypeStruct((N, 1), jnp.float32),
          jax.ShapeDtypeStruct((N, 1), jnp.float32),
      ],
  )(p, hh, dis, b, wc, bc, wr, br)


def kernel(x, edge_index, W1, b1, g1, be1, W2, b2, g2, be2, W3, b3,
           Wc, bc, Wr, br):
  src = edge_index[0].astype(jnp.int32)
  dst = edge_index[1].astype(jnp.int32)
  pad = E_PAD - E
  srcp = jnp.concatenate([src, jnp.zeros((pad,), jnp.int32)])
  dstp = jnp.concatenate([dst, jnp.full((pad,), N, jnp.int32)])
  edges = jnp.concatenate(
      [srcp.reshape(R, 1, CHUNK), dstp.reshape(R, 1, CHUNK)], axis=1)

  ones = jnp.ones((CHUNK,), jnp.float32)
  z1 = jnp.zeros((NP_D,), jnp.float32)
  z16 = jnp.zeros((NP_, 16), jnp.float32)
  z8 = jnp.zeros((NP_, 8), jnp.float32)

  deg = _deg(edges, ones, z1)
  dtot = deg[0, 0, :N] + deg[1, 0, :N]
  dis = lax.rsqrt(dtot + 1.0).reshape(N, 1)

  b1r = b1.reshape(1, 32)
  g1r = g1.reshape(1, 32)
  be1r = be1.reshape(1, 32)
  b2r = b2.reshape(1, 16)
  g2r = g2.reshape(1, 16)
  be2r = be2.reshape(1, 16)
  b3r = b3.reshape(1, 8)
  bcr = bc.reshape(1, 1)
  brr = br.reshape(1, 1)

  ha, hb = _k_pre(x, W1, dis)
  aa = _agg16(edges, ha, z16)
  ab = _agg16(edges, hb, z16)
  o1, st1 = _k_mid1(aa, ha, ab, hb, dis, b1r)
  h2 = _k_post(o1, st1, g1r, be1r, dis, W2, 32, 16)
  a2 = _agg16(edges, h2, z16)
  o2, st2 = _k_mid(a2, h2, dis, b2r, 16)
  h3 = _k_post(o2, st2, g2r, be2r, dis, W3, 16, 8)
  a3 = _agg8(edges, h3, z8)
  p, r = _k_fin(a3, h3, dis, b3r, Wc, bcr, Wr, brr)
  return (p, r)
